# speedup vs baseline: 1.0208x; 1.0208x over previous
import functools

import jax
import jax.numpy as jnp
from jax import lax
from jax.experimental import pallas as pl
from jax.experimental.pallas import tpu as pltpu
from jax.experimental.pallas import tpu_sc as plsc

_VOCAB = 100000
_D = 768
_B = 4
_S = 8192
_LANES = 16

_info = plsc.get_sparse_core_info()
_NC = _info.num_cores
_NS = _info.num_subcores
_NW = _NC * _NS
_S_PER_W = _S // _NW
_CHUNK = 32
_N_CHUNKS = _S_PER_W // _CHUNK
_ROW_VREGS = _D // _LANES


def _emb_body(ids_hbm, wte_hbm, wpe_hbm, out_hbm, idx_v, wpe_v, rows_v, sem):
    wid = lax.axis_index("s") * _NC + lax.axis_index("c")
    s0 = wid * _S_PER_W

    def chunk_body(k, carry):
        pos = s0 + k * _CHUNK
        pltpu.sync_copy(wpe_hbm.at[pl.ds(pos, _CHUNK)], wpe_v)
        for b in range(_B):
            pltpu.sync_copy(ids_hbm.at[b, pl.ds(pos, _CHUNK)], idx_v)
            for i in range(_CHUNK // _LANES):
                sl = pl.ds(i * _LANES, _LANES)
                idx_v[sl] = jnp.minimum(idx_v[sl], _VOCAB - 1)
            pltpu.async_copy(wte_hbm.at[idx_v], rows_v, sem).wait()

            def row_body(r, c2):
                for c in range(_ROW_VREGS):
                    sl = pl.ds(c * _LANES, _LANES)
                    rows_v[r, sl] = rows_v[r, sl] + wpe_v[r, sl]
                return c2

            lax.fori_loop(0, _CHUNK, row_body, 0)
            pltpu.sync_copy(rows_v, out_hbm.at[b, pl.ds(pos, _CHUNK)])
        return carry

    lax.fori_loop(0, _N_CHUNKS, chunk_body, 0)


def kernel(inputs, wte, wpe):
    mesh = plsc.VectorSubcoreMesh(core_axis_name="c", subcore_axis_name="s")
    f = pl.kernel(
        _emb_body,
        mesh=mesh,
        out_type=jax.ShapeDtypeStruct((_B, _S, _D), jnp.float32),
        scratch_types=[
            pltpu.VMEM((_CHUNK,), jnp.int32),
            pltpu.VMEM((_CHUNK, _D), jnp.float32),
            pltpu.VMEM((_CHUNK, _D), jnp.float32),
            pltpu.SemaphoreType.DMA,
        ],
    )
    return f(inputs, wte, wpe)


# speedup vs baseline: 1.6158x; 1.5829x over previous
import jax
import jax.numpy as jnp
from jax import lax
from jax.experimental import pallas as pl
from jax.experimental.pallas import tpu as pltpu
from jax.experimental.pallas import tpu_sc as plsc

_VOCAB = 100000
_D = 768
_B = 4
_S = 8192
_LANES = 16

_info = plsc.get_sparse_core_info()
_NC = _info.num_cores
_NS = _info.num_subcores
_NW = _NC * _NS
_S_PER_W = _S // _NW
_CHUNK = 32
_N_CHUNKS = _S_PER_W // _CHUNK
_PAIRS = _N_CHUNKS // 2
_ROW_VREGS = _D // _LANES


def _emb_body(ids_hbm, wte_hbm, wpe_hbm, out_hbm,
              idx_tmp, idx_all, rb0, rb1, wb0, wb1,
              sg0, sg1, so0, so1, sw0, sw1):
    wid = lax.axis_index("s") * _NC + lax.axis_index("c")
    s0 = wid * _S_PER_W

    for b in range(_B):
        pltpu.sync_copy(ids_hbm.at[b, pl.ds(s0, _S_PER_W)], idx_tmp.at[b])
    for k in range(_N_CHUNKS):
        for b in range(_B):
            for i in range(_CHUNK // _LANES):
                src = pl.ds(k * _CHUNK + i * _LANES, _LANES)
                dst = pl.ds(i * _LANES, _LANES)
                idx_all[k, b, dst] = jnp.minimum(idx_tmp[b, src], _VOCAB - 1)

    rbufs = (rb0, rb1)
    wbufs = (wb0, wb1)
    sgs = (sg0, sg1)
    sos = (so0, so1)
    sws = (sw0, sw1)

    def g_start(k, b, x):
        pltpu.async_copy(wte_hbm.at[idx_all.at[k, b]], rbufs[x], sgs[x])

    def g_wait(x):
        pltpu.make_async_copy(
            wte_hbm.at[pl.ds(0, _CHUNK)], rbufs[x], sgs[x]).wait()

    def o_start(k, b, x):
        pltpu.async_copy(
            rbufs[x], out_hbm.at[b, pl.ds(s0 + k * _CHUNK, _CHUNK)], sos[x])

    def o_wait(x):
        pltpu.make_async_copy(
            rbufs[x], out_hbm.at[0, pl.ds(0, _CHUNK)], sos[x]).wait()

    def w_start(k, p):
        pltpu.async_copy(
            wpe_hbm.at[pl.ds(s0 + k * _CHUNK, _CHUNK)], wbufs[p], sws[p])

    def w_wait(p):
        pltpu.make_async_copy(
            wpe_hbm.at[pl.ds(0, _CHUNK)], wbufs[p], sws[p]).wait()

    def vadd(x, p):
        def row_body(r, c2):
            for c in range(_ROW_VREGS):
                sl = pl.ds(c * _LANES, _LANES)
                rbufs[x][r, sl] = rbufs[x][r, sl] + wbufs[p][r, sl]
            return c2
        lax.fori_loop(0, _CHUNK, row_body, 0)

    w_start(0, 0)
    g_start(0, 0, 0)

    def pair_body(j, carry):
        for par in range(2):
            k = 2 * j + par
            for b in range(_B):
                cur = b % 2
                nxt = 1 - cur
                if (par, b) == (0, 0):
                    @pl.when(j > 0)
                    def _():
                        o_wait(nxt)
                else:
                    o_wait(nxt)
                b_next = (b + 1) % _B
                k_next = k + 1 if b == _B - 1 else k
                if (par, b) == (1, _B - 1):
                    @pl.when(j < _PAIRS - 1)
                    def _():
                        g_start(k_next, b_next, nxt)
                else:
                    g_start(k_next, b_next, nxt)
                if b == 0:
                    w_wait(par)
                    if par == 0:
                        w_start(k + 1, 1)
                    else:
                        @pl.when(j < _PAIRS - 1)
                        def _():
                            w_start(k + 1, 0)
                g_wait(cur)
                vadd(cur, par)
                o_start(k, b, cur)
        return carry

    lax.fori_loop(0, _PAIRS, pair_body, 0)
    o_wait(1)


def kernel(inputs, wte, wpe):
    mesh = plsc.VectorSubcoreMesh(core_axis_name="c", subcore_axis_name="s")
    f = pl.kernel(
        _emb_body,
        mesh=mesh,
        out_type=jax.ShapeDtypeStruct((_B, _S, _D), jnp.float32),
        scratch_types=[
            pltpu.VMEM((_B, _S_PER_W), jnp.int32),
            pltpu.VMEM((_N_CHUNKS, _B, _CHUNK), jnp.int32),
            pltpu.VMEM((_CHUNK, _D), jnp.float32),
            pltpu.VMEM((_CHUNK, _D), jnp.float32),
            pltpu.VMEM((_CHUNK, _D), jnp.float32),
            pltpu.VMEM((_CHUNK, _D), jnp.float32),
            pltpu.SemaphoreType.DMA,
            pltpu.SemaphoreType.DMA,
            pltpu.SemaphoreType.DMA,
            pltpu.SemaphoreType.DMA,
            pltpu.SemaphoreType.DMA,
            pltpu.SemaphoreType.DMA,
        ],
    )
    return f(inputs, wte, wpe)


# speedup vs baseline: 1.7349x; 1.0737x over previous
import jax
import jax.numpy as jnp
from jax import lax
from jax.experimental import pallas as pl
from jax.experimental.pallas import tpu as pltpu
from jax.experimental.pallas import tpu_sc as plsc

_VOCAB = 100000
_D = 768
_B = 4
_S = 8192
_LANES = 16

_info = plsc.get_sparse_core_info()
_NC = _info.num_cores
_NS = _info.num_subcores
_NW = _NC * _NS
_S_PER_W = _S // _NW
_CHUNK = 16
_N_CHUNKS = _S_PER_W // _CHUNK
_PAIRS = _N_CHUNKS // 2
_ROW_VREGS = _D // _LANES
_HALF = _ROW_VREGS // 2


def _emb_body(ids_hbm, wte_hbm, wpe_hbm, out_hbm,
              idx_tmp, idx_all, rb0, rb1, wb,
              sg0, sg1, so0, so1, sw):
    wid = lax.axis_index("s") * _NC + lax.axis_index("c")
    s0 = wid * _S_PER_W

    for b in range(_B):
        pltpu.sync_copy(ids_hbm.at[b, pl.ds(s0, _S_PER_W)], idx_tmp.at[b])
    for k in range(_N_CHUNKS):
        for b in range(_B):
            idx_all[k, b, pl.ds(0, _LANES)] = jnp.minimum(
                idx_tmp[b, pl.ds(k * _CHUNK, _LANES)], _VOCAB - 1)

    rbufs = (rb0, rb1)
    sgs = (sg0, sg1)
    sos = (so0, so1)

    def g_start(k, x):
        for b in range(_B):
            pltpu.async_copy(
                wte_hbm.at[idx_all.at[k, b]],
                rbufs[x].at[pl.ds(b * _CHUNK, _CHUNK)], sgs[x])

    def g_wait(x):
        pltpu.make_async_copy(
            wte_hbm.at[pl.ds(0, _B * _CHUNK)], rbufs[x], sgs[x]).wait()

    def o_start(k, x):
        for b in range(_B):
            pltpu.async_copy(
                rbufs[x].at[pl.ds(b * _CHUNK, _CHUNK)],
                out_hbm.at[b, pl.ds(s0 + k * _CHUNK, _CHUNK)], sos[x])

    def o_wait(x):
        pltpu.make_async_copy(
            rbufs[x], out_hbm.at[0, pl.ds(0, _B * _CHUNK)], sos[x]).wait()

    def w_start(k):
        pltpu.async_copy(
            wpe_hbm.at[pl.ds(s0 + k * _CHUNK, _CHUNK)], wb, sw)

    def w_wait():
        pltpu.make_async_copy(
            wpe_hbm.at[pl.ds(0, _CHUNK)], wb, sw).wait()

    def vadd(x):
        rb = rbufs[x]

        def row_body(r, c2):
            for h in range(2):
                w = [wb[r, pl.ds((h * _HALF + c) * _LANES, _LANES)]
                     for c in range(_HALF)]
                for b in range(_B):
                    for c in range(_HALF):
                        sl = pl.ds((h * _HALF + c) * _LANES, _LANES)
                        row = b * _CHUNK + r
                        rb[row, sl] = rb[row, sl] + w[c]
            return c2
        lax.fori_loop(0, _CHUNK, row_body, 0)

    w_start(0)
    g_start(0, 0)

    def pair_body(j, carry):
        for par in range(2):
            k = 2 * j + par
            cur = par
            nxt = 1 - par
            if par == 0:
                @pl.when(j > 0)
                def _():
                    o_wait(nxt)
            else:
                o_wait(nxt)
            if par == 1:
                @pl.when(j < _PAIRS - 1)
                def _():
                    g_start(k + 1, nxt)
            else:
                g_start(k + 1, nxt)
            w_wait()
            g_wait(cur)
            vadd(cur)
            @pl.when(k + 1 < _N_CHUNKS)
            def _():
                w_start(k + 1)
            o_start(k, cur)
        return carry

    lax.fori_loop(0, _PAIRS, pair_body, 0)
    o_wait(1)


def kernel(inputs, wte, wpe):
    mesh = plsc.VectorSubcoreMesh(core_axis_name="c", subcore_axis_name="s")
    f = pl.kernel(
        _emb_body,
        mesh=mesh,
        out_type=jax.ShapeDtypeStruct((_B, _S, _D), jnp.float32),
        scratch_types=[
            pltpu.VMEM((_B, _S_PER_W), jnp.int32),
            pltpu.VMEM((_N_CHUNKS, _B, _CHUNK), jnp.int32),
            pltpu.VMEM((_B * _CHUNK, _D), jnp.float32),
            pltpu.VMEM((_B * _CHUNK, _D), jnp.float32),
            pltpu.VMEM((_CHUNK, _D), jnp.float32),
            pltpu.SemaphoreType.DMA,
            pltpu.SemaphoreType.DMA,
            pltpu.SemaphoreType.DMA,
            pltpu.SemaphoreType.DMA,
            pltpu.SemaphoreType.DMA,
        ],
    )
    return f(inputs, wte, wpe)


# speedup vs baseline: 1.8327x; 1.0564x over previous
import jax
import jax.numpy as jnp
from jax import lax
from jax.experimental import pallas as pl
from jax.experimental.pallas import tpu as pltpu
from jax.experimental.pallas import tpu_sc as plsc

_VOCAB = 100000
_D = 768
_B = 4
_S = 8192
_LANES = 16

_info = plsc.get_sparse_core_info()
_NC = _info.num_cores
_NS = _info.num_subcores
_NW = _NC * _NS
_S_PER_W = _S // _NW
_CHUNK = 8
_N_CHUNKS = _S_PER_W // _CHUNK
_QUADS = _N_CHUNKS // 4
_ROW_VREGS = _D // _LANES
_HALF = _ROW_VREGS // 2


def _emb_body(ids_hbm, wte_hbm, wpe_hbm, out_hbm,
              idx_tmp, idx_all, rb0, rb1, rb2, rb3, wb0, wb1,
              sg0, sg1, sg2, sg3, so0, so1, so2, so3, sw0, sw1):
    wid = lax.axis_index("s") * _NC + lax.axis_index("c")
    s0 = wid * _S_PER_W

    for b in range(_B):
        pltpu.sync_copy(ids_hbm.at[b, pl.ds(s0, _S_PER_W)], idx_tmp.at[b])
    for b in range(_B):
        for j in range(_S_PER_W // _LANES):
            idx_all[b, j, pl.ds(0, _LANES)] = jnp.minimum(
                idx_tmp[b, pl.ds(j * _LANES, _LANES)], _VOCAB - 1)

    rbufs = (rb0, rb1, rb2, rb3)
    wbufs = (wb0, wb1)
    sgs = (sg0, sg1, sg2, sg3)
    sos = (so0, so1, so2, so3)
    sws = (sw0, sw1)

    def g_start(k, x):
        for b in range(_B):
            pltpu.async_copy(
                wte_hbm.at[idx_all.at[b, k // 2, pl.ds((k % 2) * _CHUNK,
                                                       _CHUNK)]],
                rbufs[x].at[pl.ds(b * _CHUNK, _CHUNK)], sgs[x])

    def g_wait(x):
        pltpu.make_async_copy(
            wte_hbm.at[pl.ds(0, _B * _CHUNK)], rbufs[x], sgs[x]).wait()

    def o_start(k, x):
        for b in range(_B):
            pltpu.async_copy(
                rbufs[x].at[pl.ds(b * _CHUNK, _CHUNK)],
                out_hbm.at[b, pl.ds(s0 + k * _CHUNK, _CHUNK)], sos[x])

    def o_wait(x):
        pltpu.make_async_copy(
            rbufs[x], out_hbm.at[0, pl.ds(0, _B * _CHUNK)], sos[x]).wait()

    def w_start(k, p):
        pltpu.async_copy(
            wpe_hbm.at[pl.ds(s0 + k * _CHUNK, _CHUNK)], wbufs[p], sws[p])

    def w_wait(p):
        pltpu.make_async_copy(
            wpe_hbm.at[pl.ds(0, _CHUNK)], wbufs[p], sws[p]).wait()

    def vadd(x, p):
        rb = rbufs[x]
        wb = wbufs[p]

        def row_body(r, c2):
            for h in range(2):
                w = [wb[r, pl.ds((h * _HALF + c) * _LANES, _LANES)]
                     for c in range(_HALF)]
                for b in range(_B):
                    for c in range(_HALF):
                        sl = pl.ds((h * _HALF + c) * _LANES, _LANES)
                        row = b * _CHUNK + r
                        rb[row, sl] = rb[row, sl] + w[c]
            return c2
        lax.fori_loop(0, _CHUNK, row_body, 0)

    w_start(0, 0)
    w_start(1, 1)
    g_start(0, 0)

    def quad_body(j, carry):
        for q in range(4):
            k = 4 * j + q
            cur = q
            nxt = (q + 1) % 4
            wp = q % 2
            if q < 3:
                @pl.when(j > 0)
                def _():
                    o_wait(nxt)
            else:
                o_wait(nxt)
            if q < 3:
                g_start(k + 1, nxt)
            else:
                @pl.when(j < _QUADS - 1)
                def _():
                    g_start(k + 1, nxt)
            w_wait(wp)
            g_wait(cur)
            vadd(cur, wp)
            if q < 2:
                w_start(k + 2, wp)
            else:
                @pl.when(j < _QUADS - 1)
                def _():
                    w_start(k + 2, wp)
            o_start(k, cur)
        return carry

    lax.fori_loop(0, _QUADS, quad_body, 0)
    o_wait(1)
    o_wait(2)
    o_wait(3)


def kernel(inputs, wte, wpe):
    mesh = plsc.VectorSubcoreMesh(core_axis_name="c", subcore_axis_name="s")
    f = pl.kernel(
        _emb_body,
        mesh=mesh,
        out_type=jax.ShapeDtypeStruct((_B, _S, _D), jnp.float32),
        scratch_types=[
            pltpu.VMEM((_B, _S_PER_W), jnp.int32),
            pltpu.VMEM((_B, _S_PER_W // _LANES, _LANES), jnp.int32),
            pltpu.VMEM((_B * _CHUNK, _D), jnp.float32),
            pltpu.VMEM((_B * _CHUNK, _D), jnp.float32),
            pltpu.VMEM((_B * _CHUNK, _D), jnp.float32),
            pltpu.VMEM((_B * _CHUNK, _D), jnp.float32),
            pltpu.VMEM((_CHUNK, _D), jnp.float32),
            pltpu.VMEM((_CHUNK, _D), jnp.float32),
            pltpu.SemaphoreType.DMA,
            pltpu.SemaphoreType.DMA,
            pltpu.SemaphoreType.DMA,
            pltpu.SemaphoreType.DMA,
            pltpu.SemaphoreType.DMA,
            pltpu.SemaphoreType.DMA,
            pltpu.SemaphoreType.DMA,
            pltpu.SemaphoreType.DMA,
            pltpu.SemaphoreType.DMA,
            pltpu.SemaphoreType.DMA,
        ],
    )
    return f(inputs, wte, wpe)
